# Initial kernel scaffold; baseline (speedup 1.0000x reference)
import jax
import jax.numpy as jnp
from jax.experimental import pallas as pl


def kernel(x, edge_index, batch, Wm, bm, Wo1, bo1, Wo2, bo2, Wo2_last, bo2_last):
    raise NotImplementedError("write your pallas kernel here")


# speedup vs baseline: 10.9805x; 10.9805x over previous
import functools

import jax
import jax.numpy as jnp
from jax import lax
from jax.experimental import pallas as pl
from jax.experimental.pallas import tpu as pltpu
from jax.experimental.pallas import tpu_sc as plsc

_N = 10000
_F = 128
_E = 320000
_NC, _NS = 2, 16
_NW = _NC * _NS
_EB = 128
_NB = 80
_EPAD = _NW * _NB * _EB
_NPAD = 10240
_RPS = _NPAD // _NS
_CB = 16
_NCH = _NB // _CB
_BM = 1000
_GRID = _N // _BM


def _sc_segsum(with_deg: bool):
    mesh = plsc.VectorSubcoreMesh(
        core_axis_name="c", subcore_axis_name="s",
        num_cores=_NC, num_subcores=_NS)
    out_type = [jax.ShapeDtypeStruct((_NC, _NPAD, _F), jnp.float32)]
    scratch = [
        pltpu.VMEM((_CB, _EB), jnp.int32),
        pltpu.VMEM((_CB, _EB), jnp.int32),
        pltpu.VMEM((_EB, _F), jnp.float32),
        pltpu.VMEM((_EB, _F), jnp.float32),
        pltpu.VMEM_SHARED((_NPAD, _F), jnp.float32),
        pltpu.SemaphoreType.DMA,
        pltpu.SemaphoreType.DMA,
    ]
    if with_deg:
        out_type.append(jax.ShapeDtypeStruct((_NC, _NPAD), jnp.float32))
        scratch += [
            pltpu.VMEM((_EB,), jnp.float32),
            pltpu.VMEM_SHARED((_NPAD,), jnp.float32),
        ]

    def body(m, rows, cols, *refs):
        if with_deg:
            (out, deg_out, row_v, col_v, bufa, bufb, agg, sema, semb,
             ones_v, deg_sh) = refs
        else:
            out, row_v, col_v, bufa, bufb, agg, sema, semb = refs
        c = lax.axis_index("c")
        s = lax.axis_index("s")
        wid = s * _NC + c

        def zero_row(i, _):
            for j in range(_F // 16):
                bufa[i, pl.ds(j * 16, 16)] = jnp.zeros((16,), jnp.float32)
            return 0
        lax.fori_loop(0, _EB, zero_row, 0)
        for k in range(_RPS // _EB):
            pltpu.sync_copy(bufa, agg.at[pl.ds(s * _RPS + k * _EB, _EB)])
        if with_deg:
            for k in range(_RPS // _EB):
                pltpu.sync_copy(bufa.at[0],
                                deg_sh.at[pl.ds(s * _RPS + k * _EB, _EB)])
            for j in range(_EB // 16):
                ones_v[pl.ds(j * 16, 16)] = jnp.ones((16,), jnp.float32)

        def gstart(j, buf, sem):
            pltpu.async_copy(m.at[row_v.at[j]], buf, sem)

        def gwait(j, buf, sem):
            pltpu.make_async_copy(m.at[row_v.at[j]], buf, sem).wait()

        def consume(j, buf):
            pltpu.sync_copy(buf, agg.at[col_v.at[j]], add=True)
            if with_deg:
                pltpu.sync_copy(ones_v, deg_sh.at[col_v.at[j]], add=True)

        plsc.subcore_barrier()
        for ch in range(_NCH):
            base = wid * _NB + ch * _CB
            pltpu.sync_copy(rows.at[pl.ds(base, _CB)], row_v)
            pltpu.sync_copy(cols.at[pl.ds(base, _CB)], col_v)
            gstart(0, bufa, sema)

            def step(g, _):
                j0 = g * 2
                gstart(j0 + 1, bufb, semb)
                gwait(j0, bufa, sema)
                consume(j0, bufa)
                gstart(jnp.minimum(j0 + 2, _CB - 1), bufa, sema)
                gwait(j0 + 1, bufb, semb)
                consume(j0 + 1, bufb)
                return 0
            lax.fori_loop(0, _CB // 2, step, 0)
            gwait(_CB - 1, bufa, sema)

        plsc.subcore_barrier()
        pltpu.sync_copy(agg.at[pl.ds(s * _RPS, _RPS)],
                        out.at[c, pl.ds(s * _RPS, _RPS)])
        if with_deg:
            pltpu.sync_copy(deg_sh.at[pl.ds(s * _RPS, _RPS)],
                            deg_out.at[c, pl.ds(s * _RPS, _RPS)])

    return pl.kernel(body, out_type=out_type, mesh=mesh,
                     scratch_types=scratch)


def _tc_msg(h, w, b):
    def body(h_ref, w_ref, b_ref, o_ref):
        o_ref[...] = jnp.maximum(
            jnp.dot(h_ref[...], w_ref[...],
                    preferred_element_type=jnp.float32) + b_ref[...], 0.0)
    return pl.pallas_call(
        body,
        grid=(_GRID,),
        in_specs=[pl.BlockSpec((_BM, _F), lambda i: (i, 0)),
                  pl.BlockSpec((_F, _F), lambda i: (0, 0)),
                  pl.BlockSpec((1, _F), lambda i: (0, 0))],
        out_specs=pl.BlockSpec((_BM, _F), lambda i: (i, 0)),
        out_shape=jax.ShapeDtypeStruct((_N, _F), jnp.float32),
    )(h, w, b)


def _tc_post(h, P, d0, d1, wo1, bo1, wo2, bo2, wm_n, bm_n,
             residual: bool, kout: int, with_next: bool):
    def body(h_ref, p0_ref, p1_ref, d0_ref, d1_ref, wo1_ref, bo1_ref,
             wo2_ref, bo2_ref, *rest):
        if with_next:
            wm_ref, bm_ref, hn_ref, mn_ref = rest
        else:
            (hn_ref,) = rest
        deg = jnp.maximum(d0_ref[...] + d1_ref[...], 1.0)
        aggn = (p0_ref[0] + p1_ref[0]) / deg
        fx = h_ref[...] - aggn
        cat = jnp.concatenate([fx, aggn], axis=1)
        hid = jnp.maximum(
            jnp.dot(cat, wo1_ref[...],
                    preferred_element_type=jnp.float32) + bo1_ref[...], 0.0)
        out = jnp.dot(hid, wo2_ref[...],
                      preferred_element_type=jnp.float32) + bo2_ref[...]
        if residual:
            out = out + h_ref[...]
        hn_ref[...] = out
        if with_next:
            mn_ref[...] = jnp.maximum(
                jnp.dot(out, wm_ref[...],
                        preferred_element_type=jnp.float32) + bm_ref[...], 0.0)

    in_specs = [
        pl.BlockSpec((_BM, _F), lambda i: (i, 0)),
        pl.BlockSpec((1, _BM, _F), lambda i: (0, i, 0)),
        pl.BlockSpec((1, _BM, _F), lambda i: (1, i, 0)),
        pl.BlockSpec((_BM, 1), lambda i: (i, 0)),
        pl.BlockSpec((_BM, 1), lambda i: (i, 0)),
        pl.BlockSpec((2 * _F, 2 * _F), lambda i: (0, 0)),
        pl.BlockSpec((1, 2 * _F), lambda i: (0, 0)),
        pl.BlockSpec((2 * _F, kout), lambda i: (0, 0)),
        pl.BlockSpec((1, kout), lambda i: (0, 0)),
    ]
    args = [h, P, P, d0, d1, wo1, bo1.reshape(1, 2 * _F),
            wo2, bo2.reshape(1, kout)]
    out_shape = [jax.ShapeDtypeStruct((_N, kout), jnp.float32)]
    out_specs = [pl.BlockSpec((_BM, kout), lambda i: (i, 0))]
    if with_next:
        in_specs += [pl.BlockSpec((_F, _F), lambda i: (0, 0)),
                     pl.BlockSpec((1, _F), lambda i: (0, 0))]
        args += [wm_n, bm_n.reshape(1, _F)]
        out_shape.append(jax.ShapeDtypeStruct((_N, _F), jnp.float32))
        out_specs.append(pl.BlockSpec((_BM, _F), lambda i: (i, 0)))
    res = pl.pallas_call(
        body,
        grid=(_GRID,),
        in_specs=in_specs,
        out_specs=out_specs,
        out_shape=out_shape,
    )(*args)
    return res if with_next else res[0]


def kernel(x, edge_index, batch, Wm, bm, Wo1, bo1, Wo2, bo2,
           Wo2_last, bo2_last):
    row = edge_index[0]
    col = edge_index[1]
    npad = _EPAD - _E
    ar = lax.iota(jnp.int32, npad)
    row_p = jnp.concatenate([row, ar % 16]).reshape(_NW * _NB, _EB)
    col_p = jnp.concatenate([col, _N + ar % (_NPAD - _N)]).reshape(
        _NW * _NB, _EB)

    sc_deg = _sc_segsum(True)
    sc = _sc_segsum(False)

    m0 = _tc_msg(x, Wm[0], bm[0].reshape(1, _F))
    P0, D = sc_deg(m0, row_p, col_p)
    d0 = D[0, :_N].reshape(_N, 1)
    d1 = D[1, :_N].reshape(_N, 1)

    h1, m1 = _tc_post(x, P0, d0, d1, Wo1[0], bo1[0], Wo2[0], bo2[0],
                      Wm[1], bm[1], residual=False, kout=_F, with_next=True)
    (P1,) = sc(m1, row_p, col_p)
    h2, m2 = _tc_post(h1, P1, d0, d1, Wo1[1], bo1[1], Wo2[1], bo2[1],
                      Wm[2], bm[2], residual=True, kout=_F, with_next=True)
    (P2,) = sc(m2, row_p, col_p)
    h3, m3 = _tc_post(h2, P2, d0, d1, Wo1[2], bo1[2], Wo2[2], bo2[2],
                      Wm[3], bm[3], residual=True, kout=_F, with_next=True)
    (P3,) = sc(m3, row_p, col_p)
    coords = _tc_post(h3, P3, d0, d1, Wo1[3], bo1[3], Wo2_last, bo2_last,
                      None, None, residual=False, kout=2, with_next=False)
    return coords
